# Initial kernel scaffold; baseline (speedup 1.0000x reference)
import jax
import jax.numpy as jnp
from jax.experimental import pallas as pl


def kernel(inputs, pc_coords0, pc_coords1, pc_feat1, pc_coords2, pc_feat2, pc_coords3, pc_feat3):
    raise NotImplementedError("write your pallas kernel here")


# speedup vs baseline: 23.6296x; 23.6296x over previous
import functools

import jax
import jax.numpy as jnp
from jax import lax
from jax.experimental import pallas as pl

K = 8


def _stage_body(x_ref, y_ref, f_ref, o_ref, *, M: int):
    x = x_ref[0]
    y = y_ref[0]
    xy = lax.dot_general(x, y, (((1,), (0,)), ((), ())),
                         preferred_element_type=jnp.float32)
    yy = jnp.sum(y * y, axis=0)
    key = yy[None, :] - 2.0 * xy
    iota = lax.broadcasted_iota(jnp.int32, key.shape, 1)
    w = jnp.zeros_like(key)
    for _ in range(K):
        rm = jnp.min(key, axis=1, keepdims=True)
        eq = key == rm
        idxm = jnp.min(jnp.where(eq, iota, M), axis=1, keepdims=True)
        sel = iota == idxm
        w = w + sel.astype(jnp.float32)
        key = jnp.where(sel, jnp.inf, key)
    cm = lax.dot_general(w, y, (((1,), (1,)), ((), ())),
                         preferred_element_type=jnp.float32)
    if f_ref is None:
        out = cm
    else:
        f = f_ref[0]
        fm = lax.dot_general(w, f, (((1,), (1,)), ((), ())),
                             preferred_element_type=jnp.float32)
        out = jnp.concatenate([cm, fm], axis=1)
    o_ref[0] = out * (1.0 / K)


def _stage_means(x, y, f, block_n: int = 512):
    B, N, _ = x.shape
    M = y.shape[2]
    Df = 0 if f is None else f.shape[1]
    grid = (B, N // block_n)
    in_specs = [
        pl.BlockSpec((1, block_n, 3), lambda b, n: (b, n, 0)),
        pl.BlockSpec((1, 3, M), lambda b, n: (b, 0, 0)),
    ]
    args = [x, y]
    if f is None:
        body = functools.partial(_stage_body, f_ref=None, M=M)

        def wrapped(x_ref, y_ref, o_ref):
            body(x_ref, y_ref, o_ref=o_ref)
    else:
        in_specs.append(pl.BlockSpec((1, Df, M), lambda b, n: (b, 0, 0)))
        args.append(f)

        def wrapped(x_ref, y_ref, f_ref, o_ref):
            _stage_body(x_ref, y_ref, f_ref, o_ref, M=M)

    return pl.pallas_call(
        wrapped,
        grid=grid,
        in_specs=in_specs,
        out_specs=pl.BlockSpec((1, block_n, 3 + Df), lambda b, n: (b, n, 0)),
        out_shape=jax.ShapeDtypeStruct((B, N, 3 + Df), jnp.float32),
    )(*args)


def kernel(inputs, pc_coords0, pc_coords1, pc_feat1, pc_coords2, pc_feat2,
           pc_coords3, pc_feat3):
    s0 = _stage_means(inputs, pc_coords0, None)
    s1 = _stage_means(inputs, pc_coords1, pc_feat1)
    s2 = _stage_means(inputs, pc_coords2, pc_feat2)
    s3 = _stage_means(inputs, pc_coords3, pc_feat3)
    return jnp.concatenate([inputs, s0, s1, s2, s3], axis=2)
